# Initial kernel scaffold; baseline (speedup 1.0000x reference)
import jax
import jax.numpy as jnp
from jax.experimental import pallas as pl


def kernel(x, target):
    raise NotImplementedError("write your pallas kernel here")


# speedup vs baseline: 2.2283x; 2.2283x over previous
import functools
import math

import jax
import jax.numpy as jnp
from jax import lax
from jax.experimental import pallas as pl
from jax.experimental.pallas import tpu as pltpu
from jax.experimental.pallas import tpu_sc as plsc

ROWS = 2048
SIZE = 32000
PADDING_IDX = 0
SMOOTHING = 0.1
CONFIDENCE = 1.0 - SMOOTHING
EPS = SMOOTHING / (SIZE - 1)
C_CONST = (SIZE - 1) * EPS * math.log(EPS) + CONFIDENCE * math.log(CONFIDENCE)


RB = 256
CB = 3200
NI = ROWS // RB
NJ = SIZE // CB


def _rowsum_body(t_ref, x_ref, o_ref, acc_ref):
    i = pl.program_id(0)
    j = pl.program_id(1)

    @pl.when((i == 0) & (j == 0))
    def _init():
        acc_ref[0] = 0.0

    t = t_ref[0]
    m = (t != PADDING_IDX).astype(jnp.float32)
    xb = x_ref[...]
    part = jnp.dot(m, xb, preferred_element_type=jnp.float32)
    acc_ref[0] += jnp.sum(part)

    @pl.when((i == NI - 1) & (j == NJ - 1))
    def _fin():
        o_ref[0, 0] = acc_ref[0]


def _masked_rowsum(x, target):
    t3 = target.reshape(NI, 1, RB)
    out = pl.pallas_call(
        _rowsum_body,
        grid=(NI, NJ),
        in_specs=[
            pl.BlockSpec((1, 1, RB), lambda i, j: (i, 0, 0)),
            pl.BlockSpec((RB, CB), lambda i, j: (i, j)),
        ],
        out_specs=pl.BlockSpec((1, 1), lambda i, j: (0, 0),
                               memory_space=pltpu.SMEM),
        out_shape=jax.ShapeDtypeStruct((1, 1), jnp.float32),
        scratch_shapes=[pltpu.SMEM((1,), jnp.float32)],
    )(t3, x)
    return out[0, 0]


L = 16
NW = 32
BPW = ROWS // NW
NCHUNK = BPW // L


def _sc_gather_kernel(tgt_hbm, xflat_hbm, b_hbm, tok_hbm,
                      tgt_v, idx_v, val_v, out_v, sem):
    wid = lax.axis_index("s") * 2 + lax.axis_index("c")
    base = wid * BPW
    pltpu.sync_copy(tgt_hbm.at[pl.ds(base, BPW)], tgt_v)
    lanes = lax.iota(jnp.int32, L)
    for c in range(NCHUNK):
        t16 = tgt_v[pl.ds(c * L, L)]
        rows = base + c * L + lanes
        idx_v[pl.ds(c * L, L)] = rows * SIZE + t16
    pltpu.async_copy(xflat_hbm.at[idx_v], val_v, sem).wait()
    bacc = jnp.zeros((L,), jnp.float32)
    tacc = jnp.zeros((L,), jnp.float32)
    for c in range(NCHUNK):
        t16 = tgt_v[pl.ds(c * L, L)]
        v16 = val_v[pl.ds(c * L, L)]
        m = t16 != PADDING_IDX
        bacc = bacc + jnp.where(m, v16, 0.0)
        tacc = tacc + jnp.where(m, 1.0, 0.0)
    out_v[...] = bacc
    pltpu.sync_copy(out_v, b_hbm.at[wid])
    out_v[...] = tacc
    pltpu.sync_copy(out_v, tok_hbm.at[wid])


@functools.cache
def _make_sc_gather():
    return functools.partial(
        pl.kernel,
        mesh=plsc.VectorSubcoreMesh(core_axis_name="c", subcore_axis_name="s"),
        out_type=[
            jax.ShapeDtypeStruct((NW, L), jnp.float32),
            jax.ShapeDtypeStruct((NW, L), jnp.float32),
        ],
        scratch_types=[
            pltpu.VMEM((BPW,), jnp.int32),
            pltpu.VMEM((BPW,), jnp.int32),
            pltpu.VMEM((BPW,), jnp.float32),
            pltpu.VMEM((L,), jnp.float32),
            pltpu.SemaphoreType.DMA,
        ],
    )(_sc_gather_kernel)


def kernel(x, target):
    target = target.astype(jnp.int32)
    a_sum = _masked_rowsum(x, target)
    b_parts, tok_parts = _make_sc_gather()(target, x.reshape(-1))
    b_sum = jnp.sum(b_parts)
    tokens = jnp.sum(tok_parts)
    c32 = jnp.float32(C_CONST)
    return c32 - (jnp.float32(EPS) * a_sum
                  + jnp.float32(CONFIDENCE - EPS) * b_sum) / tokens


# speedup vs baseline: 2.4351x; 1.0928x over previous
import functools
import math

import jax
import jax.numpy as jnp
from jax import lax
from jax.experimental import pallas as pl
from jax.experimental.pallas import tpu as pltpu
from jax.experimental.pallas import tpu_sc as plsc

ROWS = 2048
SIZE = 32000
PADDING_IDX = 0
SMOOTHING = 0.1
CONFIDENCE = 1.0 - SMOOTHING
EPS = SMOOTHING / (SIZE - 1)
C_CONST = (SIZE - 1) * EPS * math.log(EPS) + CONFIDENCE * math.log(CONFIDENCE)


RB = 128
CB = SIZE
NI = ROWS // RB
NJ = SIZE // CB


def _rowsum_body(t_ref, x_ref, o_ref, acc_ref):
    i = pl.program_id(0)

    @pl.when(i == 0)
    def _init():
        acc_ref[0] = 0.0

    t = t_ref[0]
    m = (t != PADDING_IDX).astype(jnp.float32)
    xb = x_ref[...]
    part = jnp.dot(m, xb, preferred_element_type=jnp.float32)
    acc_ref[0] += jnp.sum(part)

    @pl.when(i == NI - 1)
    def _fin():
        o_ref[0, 0] = acc_ref[0]


def _masked_rowsum(x, target):
    t3 = target.reshape(NI, 1, RB)
    out = pl.pallas_call(
        _rowsum_body,
        grid=(NI,),
        in_specs=[
            pl.BlockSpec((1, 1, RB), lambda i: (i, 0, 0)),
            pl.BlockSpec((RB, CB), lambda i: (i, 0)),
        ],
        out_specs=pl.BlockSpec((1, 1), lambda i: (0, 0),
                               memory_space=pltpu.SMEM),
        out_shape=jax.ShapeDtypeStruct((1, 1), jnp.float32),
        scratch_shapes=[pltpu.SMEM((1,), jnp.float32)],
    )(t3, x)
    return out[0, 0]


L = 16
NW = 32
BPW = ROWS // NW
NCHUNK = BPW // L


def _sc_gather_kernel(tgt_hbm, xflat_hbm, b_hbm, tok_hbm,
                      tgt_v, idx_v, val_v, out_v, sem):
    wid = lax.axis_index("s") * 2 + lax.axis_index("c")
    base = wid * BPW
    pltpu.sync_copy(tgt_hbm.at[pl.ds(base, BPW)], tgt_v)
    lanes = lax.iota(jnp.int32, L)
    for c in range(NCHUNK):
        t16 = tgt_v[pl.ds(c * L, L)]
        rows = base + c * L + lanes
        idx_v[pl.ds(c * L, L)] = rows * SIZE + t16
    pltpu.async_copy(xflat_hbm.at[idx_v], val_v, sem).wait()
    bacc = jnp.zeros((L,), jnp.float32)
    tacc = jnp.zeros((L,), jnp.float32)
    for c in range(NCHUNK):
        t16 = tgt_v[pl.ds(c * L, L)]
        v16 = val_v[pl.ds(c * L, L)]
        m = t16 != PADDING_IDX
        bacc = bacc + jnp.where(m, v16, 0.0)
        tacc = tacc + jnp.where(m, 1.0, 0.0)
    out_v[...] = bacc
    pltpu.sync_copy(out_v, b_hbm.at[wid])
    out_v[...] = tacc
    pltpu.sync_copy(out_v, tok_hbm.at[wid])


@functools.cache
def _make_sc_gather():
    return functools.partial(
        pl.kernel,
        mesh=plsc.VectorSubcoreMesh(core_axis_name="c", subcore_axis_name="s"),
        out_type=[
            jax.ShapeDtypeStruct((NW, L), jnp.float32),
            jax.ShapeDtypeStruct((NW, L), jnp.float32),
        ],
        scratch_types=[
            pltpu.VMEM((BPW,), jnp.int32),
            pltpu.VMEM((BPW,), jnp.int32),
            pltpu.VMEM((BPW,), jnp.float32),
            pltpu.VMEM((L,), jnp.float32),
            pltpu.SemaphoreType.DMA,
        ],
    )(_sc_gather_kernel)


def kernel(x, target):
    target = target.astype(jnp.int32)
    a_sum = _masked_rowsum(x, target)
    b_parts, tok_parts = _make_sc_gather()(target, x.reshape(-1))
    b_sum = jnp.sum(b_parts)
    tokens = jnp.sum(tok_parts)
    c32 = jnp.float32(C_CONST)
    return c32 - (jnp.float32(EPS) * a_sum
                  + jnp.float32(CONFIDENCE - EPS) * b_sum) / tokens
